# baseline (speedup 1.0000x reference)
import functools

import jax
import jax.numpy as jnp
from jax.experimental import pallas as pl
from jax.experimental.pallas import tpu as pltpu

B = 4096
D = 1024
NZ = 256
E = 8
H = 512
C = 16
BT = 512


def _fused_body(x_ref, we_ref, be_ref, wd_ref, bd_ref, ct_ref,
                w1_ref, b1_ref, w2_ref, b2_ref,
                preds_ref, xbar_ref, q_ref):
    x = x_ref[...]
    z = jnp.dot(x, we_ref[...], preferred_element_type=jnp.float32) + be_ref[...]
    zb = z.astype(jnp.bfloat16)
    xbar_ref[...] = (
        jnp.dot(zb, wd_ref[...], preferred_element_type=jnp.float32) + bd_ref[...]
    )
    ct = ct_ref[...]
    zn = jnp.sum(z * z, axis=1, keepdims=True)
    cn = jnp.sum(ct * ct, axis=0, keepdims=True)
    cross = jnp.dot(z, ct, preferred_element_type=jnp.float32)
    d2 = zn + cn - 2.0 * cross
    qu = 1.0 / (1.0 + d2)
    q = qu / jnp.sum(qu, axis=1, keepdims=True)
    q_ref[...] = q
    h = jnp.dot(zb, w1_ref[...], preferred_element_type=jnp.float32) + b1_ref[...]
    h = jnp.maximum(h, 0.0)
    hq = (h.reshape(BT, E, H) * q[:, :, None]).reshape(BT, E * H)
    preds_ref[...] = (
        jnp.dot(hq.astype(jnp.bfloat16), w2_ref[...],
                preferred_element_type=jnp.float32)
        + jnp.dot(q, b2_ref[...], preferred_element_type=jnp.float32)
    )


@jax.jit
def kernel(x, W_enc, b_enc, W_dec, b_dec, cluster_layer, W1, b1, W2, b2):
    w1r = W1.transpose(1, 0, 2).reshape(NZ, E * H).astype(jnp.bfloat16)
    w2r = W2.reshape(E * H, C).astype(jnp.bfloat16)
    wd = W_dec.astype(jnp.bfloat16)
    b1r = b1.reshape(1, E * H)
    ct = cluster_layer.T
    be = b_enc.reshape(1, NZ)
    bd = b_dec.reshape(1, D)

    grid = (B // BT,)
    tok = lambda i: (i, 0)
    rep = lambda i: (0, 0)
    preds, x_bar, q = pl.pallas_call(
        _fused_body,
        grid=grid,
        in_specs=[
            pl.BlockSpec((BT, D), tok),
            pl.BlockSpec((D, NZ), rep),
            pl.BlockSpec((1, NZ), rep),
            pl.BlockSpec((NZ, D), rep),
            pl.BlockSpec((1, D), rep),
            pl.BlockSpec((NZ, E), rep),
            pl.BlockSpec((NZ, E * H), rep),
            pl.BlockSpec((1, E * H), rep),
            pl.BlockSpec((E * H, C), rep),
            pl.BlockSpec((E, C), rep),
        ],
        out_specs=[
            pl.BlockSpec((BT, C), tok),
            pl.BlockSpec((BT, D), tok),
            pl.BlockSpec((BT, E), tok),
        ],
        out_shape=[
            jax.ShapeDtypeStruct((B, C), jnp.float32),
            jax.ShapeDtypeStruct((B, D), jnp.float32),
            jax.ShapeDtypeStruct((B, E), jnp.float32),
        ],
        compiler_params=pltpu.CompilerParams(
            dimension_semantics=("parallel",),
        ),
    )(x, W_enc, be, wd, bd, ct, w1r, b1r, w2r, b2)
    return (preds, x_bar, q)


# speedup vs baseline: 1.3717x; 1.3717x over previous
import functools

import jax
import jax.numpy as jnp
from jax.experimental import pallas as pl
from jax.experimental.pallas import tpu as pltpu

B = 4096
D = 1024
NZ = 256
E = 8
H = 512
C = 16
BT = 512


def _fused_body(x_ref, we_ref, be_ref, wd_ref, bd_ref, ct_ref,
                w1_ref, b1_ref, w2_ref, b2_ref, s_ref,
                preds_ref, xbar_ref, q_ref):
    x = x_ref[...]
    z = jnp.dot(x, we_ref[...], preferred_element_type=jnp.float32) + be_ref[...]
    zb = z.astype(jnp.bfloat16)
    xbar_ref[...] = (
        jnp.dot(zb, wd_ref[...], preferred_element_type=jnp.float32) + bd_ref[...]
    )
    ct = ct_ref[...]
    zn = jnp.sum(z * z, axis=1, keepdims=True)
    cn = jnp.sum(ct * ct, axis=0, keepdims=True)
    cross = jnp.dot(z, ct, preferred_element_type=jnp.float32)
    d2 = zn + cn - 2.0 * cross
    qu = 1.0 / (1.0 + d2)
    q = qu / jnp.sum(qu, axis=1, keepdims=True)
    q_ref[...] = q
    h = jnp.dot(zb, w1_ref[...], preferred_element_type=jnp.float32) + b1_ref[...]
    h = jnp.maximum(h, 0.0)
    qrep = jnp.dot(q, s_ref[...], preferred_element_type=jnp.float32)
    hq = (h * qrep).astype(jnp.bfloat16)
    preds_ref[...] = (
        jnp.dot(hq, w2_ref[...], preferred_element_type=jnp.float32)
        + jnp.dot(q, b2_ref[...], preferred_element_type=jnp.float32)
    )


@jax.jit
def kernel(x, W_enc, b_enc, W_dec, b_dec, cluster_layer, W1, b1, W2, b2):
    w1r = W1.transpose(1, 0, 2).reshape(NZ, E * H).astype(jnp.bfloat16)
    w2r = W2.reshape(E * H, C).astype(jnp.bfloat16)
    wd = W_dec.astype(jnp.bfloat16)
    b1r = b1.reshape(1, E * H)
    ct = cluster_layer.T
    be = b_enc.reshape(1, NZ)
    bd = b_dec.reshape(1, D)
    sel = jnp.repeat(jnp.eye(E, dtype=jnp.float32), H, axis=1)

    grid = (B // BT,)
    tok = lambda i: (i, 0)
    rep = lambda i: (0, 0)
    preds, x_bar, q = pl.pallas_call(
        _fused_body,
        grid=grid,
        in_specs=[
            pl.BlockSpec((BT, D), tok),
            pl.BlockSpec((D, NZ), rep),
            pl.BlockSpec((1, NZ), rep),
            pl.BlockSpec((NZ, D), rep),
            pl.BlockSpec((1, D), rep),
            pl.BlockSpec((NZ, E), rep),
            pl.BlockSpec((NZ, E * H), rep),
            pl.BlockSpec((1, E * H), rep),
            pl.BlockSpec((E * H, C), rep),
            pl.BlockSpec((E, C), rep),
            pl.BlockSpec((E, E * H), rep),
        ],
        out_specs=[
            pl.BlockSpec((BT, C), tok),
            pl.BlockSpec((BT, D), tok),
            pl.BlockSpec((BT, E), tok),
        ],
        out_shape=[
            jax.ShapeDtypeStruct((B, C), jnp.float32),
            jax.ShapeDtypeStruct((B, D), jnp.float32),
            jax.ShapeDtypeStruct((B, E), jnp.float32),
        ],
        compiler_params=pltpu.CompilerParams(
            dimension_semantics=("parallel",),
        ),
    )(x, W_enc, be, wd, bd, ct, w1r, b1r, w2r, b2, sel)
    return (preds, x_bar, q)


# speedup vs baseline: 1.5050x; 1.0971x over previous
import functools

import jax
import jax.numpy as jnp
from jax.experimental import pallas as pl
from jax.experimental.pallas import tpu as pltpu

B = 4096
D = 1024
NZ = 256
E = 8
H = 512
C = 16
BT = 512


def _fused_body(x_ref, we_ref, be_ref, wd_ref, bd_ref, ct_ref,
                w1_ref, b1_ref, w2_ref, b2_ref, s_ref,
                preds_ref, xbar_ref, q_ref):
    x = x_ref[...]
    z = jnp.dot(x, we_ref[...], preferred_element_type=jnp.float32) + be_ref[...]
    zb = z.astype(jnp.bfloat16)
    xbar_ref[...] = (
        jnp.dot(zb, wd_ref[...], preferred_element_type=jnp.float32) + bd_ref[...]
    )
    ct = ct_ref[...]
    zn = jnp.sum(z * z, axis=1, keepdims=True)
    cn = jnp.sum(ct * ct, axis=0, keepdims=True)
    cross = jnp.dot(z, ct, preferred_element_type=jnp.float32)
    d2 = zn + cn - 2.0 * cross
    qu = 1.0 / (1.0 + d2)
    q = qu / jnp.sum(qu, axis=1, keepdims=True)
    q_ref[...] = q
    h = jnp.dot(zb, w1_ref[...], preferred_element_type=jnp.float32) + b1_ref[...]
    h = jnp.maximum(h, 0.0).astype(jnp.bfloat16)
    logits = jnp.dot(h, w2_ref[...], preferred_element_type=jnp.float32)
    lq = logits * jnp.dot(q, s_ref[...], preferred_element_type=jnp.float32)
    preds = jnp.dot(q, b2_ref[...], preferred_element_type=jnp.float32)
    for e in range(E):
        preds = preds + lq[:, e * C:(e + 1) * C]
    preds_ref[...] = preds


@jax.jit
def kernel(x, W_enc, b_enc, W_dec, b_dec, cluster_layer, W1, b1, W2, b2):
    w1r = W1.transpose(1, 0, 2).reshape(NZ, E * H).astype(jnp.bfloat16)
    eye = jnp.eye(E, dtype=jnp.float32)
    w2bd = (eye[:, None, :, None] * W2[:, :, None, :]).reshape(E * H, E * C)
    w2bd = w2bd.astype(jnp.bfloat16)
    wd = W_dec.astype(jnp.bfloat16)
    b1r = b1.reshape(1, E * H)
    ct = cluster_layer.T
    be = b_enc.reshape(1, NZ)
    bd = b_dec.reshape(1, D)
    sel = jnp.repeat(jnp.eye(E, dtype=jnp.float32), C, axis=1)

    grid = (B // BT,)
    tok = lambda i: (i, 0)
    rep = lambda i: (0, 0)
    preds, x_bar, q = pl.pallas_call(
        _fused_body,
        grid=grid,
        in_specs=[
            pl.BlockSpec((BT, D), tok),
            pl.BlockSpec((D, NZ), rep),
            pl.BlockSpec((1, NZ), rep),
            pl.BlockSpec((NZ, D), rep),
            pl.BlockSpec((1, D), rep),
            pl.BlockSpec((NZ, E), rep),
            pl.BlockSpec((NZ, E * H), rep),
            pl.BlockSpec((1, E * H), rep),
            pl.BlockSpec((E * H, E * C), rep),
            pl.BlockSpec((E, C), rep),
            pl.BlockSpec((E, E * C), rep),
        ],
        out_specs=[
            pl.BlockSpec((BT, C), tok),
            pl.BlockSpec((BT, D), tok),
            pl.BlockSpec((BT, E), tok),
        ],
        out_shape=[
            jax.ShapeDtypeStruct((B, C), jnp.float32),
            jax.ShapeDtypeStruct((B, D), jnp.float32),
            jax.ShapeDtypeStruct((B, E), jnp.float32),
        ],
        compiler_params=pltpu.CompilerParams(
            dimension_semantics=("parallel",),
        ),
    )(x, W_enc, be, wd, bd, ct, w1r, b1r, w2bd, b2, sel)
    return (preds, x_bar, q)
